# initial kernel scaffold (unmeasured)
import jax
import jax.numpy as jnp
from jax import lax
from jax.experimental import pallas as pl
from jax.experimental.pallas import tpu as pltpu

N_DEV = 4
M = 4096
K_SHARD = 1024
N_GLOBAL = 8192
M_CHUNK = M // N_DEV
TILE_N = 1024
N_TILES = N_GLOBAL // TILE_N


def kernel(x, w_mat):
    def body(x_ref, w_ref, o_ref, comm, send_sems, recv_sems):
        d = lax.axis_index("i")
        left = lax.rem(d + N_DEV - 1, N_DEV)
        right = lax.rem(d + 1, N_DEV)

        barrier = pltpu.get_barrier_semaphore()
        pl.semaphore_signal(
            barrier, inc=1, device_id=(left,),
            device_id_type=pl.DeviceIdType.MESH,
        )
        pl.semaphore_signal(
            barrier, inc=1, device_id=(right,),
            device_id_type=pl.DeviceIdType.MESH,
        )
        pl.semaphore_wait(barrier, 2)

        w_b = w_ref[...].astype(jnp.bfloat16)

        def partial(c):
            xa = x_ref[pl.ds(c * M_CHUNK, M_CHUNK), :].astype(jnp.bfloat16)
            return jnp.dot(xa, w_b, preferred_element_type=jnp.float32)

        comm[0, :, :] = partial(lax.rem(d + N_DEV - 1, N_DEV))
        for s in range(N_DEV - 1):
            rdma = pltpu.make_async_remote_copy(
                src_ref=comm.at[s],
                dst_ref=comm.at[s + 1],
                send_sem=send_sems.at[s],
                recv_sem=recv_sems.at[s],
                device_id=(right,),
                device_id_type=pl.DeviceIdType.MESH,
            )
            rdma.start()
            rdma.wait()
            c = lax.rem(d + 2 * N_DEV - 2 - s, N_DEV)
            comm[s + 1, :, :] = comm[s + 1, :, :] + partial(c)
        y = comm[N_DEV - 1, :, :]
        o_ref[...] = y * jax.nn.sigmoid(y)

    return pl.pallas_call(
        body,
        grid=(N_TILES,),
        in_specs=[
            pl.BlockSpec((M, K_SHARD), lambda t: (0, 0)),
            pl.BlockSpec((K_SHARD, TILE_N), lambda t: (0, t)),
        ],
        out_specs=pl.BlockSpec((M_CHUNK, TILE_N), lambda t: (0, t)),
        out_shape=jax.ShapeDtypeStruct((M_CHUNK, N_GLOBAL), jnp.float32),
        scratch_shapes=[
            pltpu.VMEM((N_DEV, M_CHUNK, TILE_N), jnp.float32),
            pltpu.SemaphoreType.DMA((N_DEV - 1,)),
            pltpu.SemaphoreType.DMA((N_DEV - 1,)),
        ],
        compiler_params=pltpu.CompilerParams(
            collective_id=0,
            dimension_semantics=("arbitrary",),
        ),
    )(x, w_mat)


# baseline (device time: 1245588 ns/iter reference)
import jax
import jax.numpy as jnp
from jax import lax
from jax.experimental import pallas as pl
from jax.experimental.pallas import tpu as pltpu

N_DEV = 4
M = 4096
K_SHARD = 1024
N_GLOBAL = 8192
M_CHUNK = M // N_DEV
TILE_N = 1024
N_TILES = N_GLOBAL // TILE_N


def kernel(x, w_mat):
    def body(x_ref, w_ref, o_ref, comm, send_sems, recv_sems):
        d = lax.axis_index("i")
        left = lax.rem(d + N_DEV - 1, N_DEV)
        right = lax.rem(d + 1, N_DEV)

        barrier = pltpu.get_barrier_semaphore()
        pl.semaphore_signal(
            barrier, inc=1, device_id=(left,),
            device_id_type=pl.DeviceIdType.MESH,
        )
        pl.semaphore_signal(
            barrier, inc=1, device_id=(right,),
            device_id_type=pl.DeviceIdType.MESH,
        )
        pl.semaphore_wait(barrier, 2)

        w_b = w_ref[...].astype(jnp.bfloat16)

        def partial(c):
            xa = x_ref[pl.ds(c * M_CHUNK, M_CHUNK), :].astype(jnp.bfloat16)
            return jnp.dot(xa, w_b, preferred_element_type=jnp.float32)

        comm[0, :, :] = partial(lax.rem(d + N_DEV - 1, N_DEV))
        for s in range(N_DEV - 1):
            rdma = pltpu.make_async_remote_copy(
                src_ref=comm.at[s],
                dst_ref=comm.at[s + 1],
                send_sem=send_sems.at[s],
                recv_sem=recv_sems.at[s],
                device_id=(right,),
                device_id_type=pl.DeviceIdType.MESH,
            )
            rdma.start()
            rdma.wait()
            c = lax.rem(d + 2 * N_DEV - 2 - s, N_DEV)
            comm[s + 1, :, :] = comm[s + 1, :, :] + partial(c)
        y = comm[N_DEV - 1, :, :]
        o_ref[...] = y * jax.nn.sigmoid(y)

    return pl.pallas_call(
        body,
        grid=(N_TILES,),
        in_specs=[
            pl.BlockSpec((M, K_SHARD), lambda t: (0, 0)),
            pl.BlockSpec((K_SHARD, TILE_N), lambda t: (0, t)),
        ],
        out_specs=pl.BlockSpec((M_CHUNK, TILE_N), lambda t: (0, t)),
        out_shape=jax.ShapeDtypeStruct((M_CHUNK, N_GLOBAL), jnp.float32),
        scratch_shapes=[
            pltpu.VMEM((N_DEV, M_CHUNK, TILE_N), jnp.float32),
            pltpu.SemaphoreType.DMA((N_DEV - 1,)),
            pltpu.SemaphoreType.DMA((N_DEV - 1,)),
        ],
        compiler_params=pltpu.CompilerParams(
            collective_id=0,
            dimension_semantics=("arbitrary",),
            vmem_limit_bytes=60 * 1024 * 1024,
        ),
    )(x, w_mat)


# device time: 441767 ns/iter; 2.8196x vs baseline; 2.8196x over previous
import jax
import jax.numpy as jnp
from jax import lax
from jax.experimental import pallas as pl
from jax.experimental.pallas import tpu as pltpu

N_DEV = 4
M = 4096
K_SHARD = 1024
N_GLOBAL = 8192
M_CHUNK = M // N_DEV
TILE_N = 1024
STEP_N = 2 * TILE_N
N_STEPS = N_GLOBAL // STEP_N


def kernel(x, w_mat):
    def body(x_ref, w_ref, o_ref, comm_a, comm_b,
             send_a, recv_a, send_b, recv_b):
        d = lax.axis_index("i")
        left = lax.rem(d + N_DEV - 1, N_DEV)
        right = lax.rem(d + 1, N_DEV)

        barrier = pltpu.get_barrier_semaphore()
        pl.semaphore_signal(
            barrier, inc=1, device_id=(left,),
            device_id_type=pl.DeviceIdType.MESH,
        )
        pl.semaphore_signal(
            barrier, inc=1, device_id=(right,),
            device_id_type=pl.DeviceIdType.MESH,
        )
        pl.semaphore_wait(barrier, 2)

        def part_a(c):
            return jnp.dot(
                x_ref[pl.ds(c * M_CHUNK, M_CHUNK), :],
                w_ref[:, :TILE_N],
                preferred_element_type=jnp.float32,
            )

        def part_b(c):
            return jnp.dot(
                x_ref[pl.ds(c * M_CHUNK, M_CHUNK), :],
                w_ref[:, TILE_N:],
                preferred_element_type=jnp.float32,
            )

        comm_a[0, :, :] = part_a(lax.rem(d + N_DEV - 1, N_DEV)).astype(
            jnp.bfloat16)
        comm_b[0, :, :] = part_b(lax.rem(d + 1, N_DEV)).astype(jnp.bfloat16)

        for s in range(N_DEV - 1):
            rdma_a = pltpu.make_async_remote_copy(
                src_ref=comm_a.at[s],
                dst_ref=comm_a.at[s + 1],
                send_sem=send_a.at[s],
                recv_sem=recv_a.at[s],
                device_id=(right,),
                device_id_type=pl.DeviceIdType.MESH,
            )
            rdma_a.start()
            rdma_b = pltpu.make_async_remote_copy(
                src_ref=comm_b.at[s],
                dst_ref=comm_b.at[s + 1],
                send_sem=send_b.at[s],
                recv_sem=recv_b.at[s],
                device_id=(left,),
                device_id_type=pl.DeviceIdType.MESH,
            )
            rdma_b.start()
            rdma_a.wait()
            ca = lax.rem(d + 2 * N_DEV - 2 - s, N_DEV)
            comm_a[s + 1, :, :] = (
                comm_a[s + 1, :, :].astype(jnp.float32) + part_a(ca)
            ).astype(jnp.bfloat16)
            rdma_b.wait()
            cb = lax.rem(d + 2 + s, N_DEV)
            comm_b[s + 1, :, :] = (
                comm_b[s + 1, :, :].astype(jnp.float32) + part_b(cb)
            ).astype(jnp.bfloat16)

        ya = comm_a[N_DEV - 1, :, :].astype(jnp.float32)
        yb = comm_b[N_DEV - 1, :, :].astype(jnp.float32)
        o_ref[:, :TILE_N] = ya * jax.nn.sigmoid(ya)
        o_ref[:, TILE_N:] = yb * jax.nn.sigmoid(yb)

    out = pl.pallas_call(
        body,
        grid=(N_STEPS,),
        in_specs=[
            pl.BlockSpec((M, K_SHARD), lambda t: (0, 0)),
            pl.BlockSpec((K_SHARD, STEP_N), lambda t: (0, t)),
        ],
        out_specs=pl.BlockSpec((M_CHUNK, STEP_N), lambda t: (0, t)),
        out_shape=jax.ShapeDtypeStruct((M_CHUNK, N_GLOBAL), jnp.float32),
        scratch_shapes=[
            pltpu.VMEM((N_DEV, M_CHUNK, TILE_N), jnp.bfloat16),
            pltpu.VMEM((N_DEV, M_CHUNK, TILE_N), jnp.bfloat16),
            pltpu.SemaphoreType.DMA((N_DEV - 1,)),
            pltpu.SemaphoreType.DMA((N_DEV - 1,)),
            pltpu.SemaphoreType.DMA((N_DEV - 1,)),
            pltpu.SemaphoreType.DMA((N_DEV - 1,)),
        ],
        compiler_params=pltpu.CompilerParams(
            collective_id=0,
            dimension_semantics=("arbitrary",),
            vmem_limit_bytes=60 * 1024 * 1024,
        ),
    )(x.astype(jnp.bfloat16), w_mat.astype(jnp.bfloat16))
    return out


# device time: 393536 ns/iter; 3.1651x vs baseline; 1.1226x over previous
import jax
import jax.numpy as jnp
from jax import lax
from jax.experimental import pallas as pl
from jax.experimental.pallas import tpu as pltpu

N_DEV = 4
M = 4096
K_SHARD = 1024
N_GLOBAL = 8192
M_CHUNK = M // N_DEV
TILE_N = 1024
STEP_N = 2 * TILE_N
N_STEPS = N_GLOBAL // STEP_N


def kernel(x, w_mat):
    def body(x_ref, w_ref, o_ref, comm_a, comm_b,
             send_a, recv_a, send_b, recv_b):
        d = lax.axis_index("i")
        left = lax.rem(d + N_DEV - 1, N_DEV)
        right = lax.rem(d + 1, N_DEV)

        barrier = pltpu.get_barrier_semaphore()
        pl.semaphore_signal(
            barrier, inc=1, device_id=(left,),
            device_id_type=pl.DeviceIdType.MESH,
        )
        pl.semaphore_signal(
            barrier, inc=1, device_id=(right,),
            device_id_type=pl.DeviceIdType.MESH,
        )
        pl.semaphore_wait(barrier, 2)

        def part_a(c):
            return jnp.dot(
                x_ref[pl.ds(c * M_CHUNK, M_CHUNK), :],
                w_ref[:, :TILE_N],
                preferred_element_type=jnp.float32,
            )

        def part_b(c):
            return jnp.dot(
                x_ref[pl.ds(c * M_CHUNK, M_CHUNK), :],
                w_ref[:, TILE_N:],
                preferred_element_type=jnp.float32,
            )

        comm_a[0, :, :] = part_a(lax.rem(d + N_DEV - 1, N_DEV)).astype(
            jnp.bfloat16)
        comm_b[0, :, :] = part_b(lax.rem(d + 1, N_DEV)).astype(jnp.bfloat16)

        for s in range(N_DEV - 1):
            rdma_a = pltpu.make_async_remote_copy(
                src_ref=comm_a.at[s],
                dst_ref=comm_a.at[s + 1],
                send_sem=send_a.at[s],
                recv_sem=recv_a.at[s],
                device_id=(right,),
                device_id_type=pl.DeviceIdType.MESH,
            )
            rdma_a.start()
            rdma_b = pltpu.make_async_remote_copy(
                src_ref=comm_b.at[s],
                dst_ref=comm_b.at[s + 1],
                send_sem=send_b.at[s],
                recv_sem=recv_b.at[s],
                device_id=(left,),
                device_id_type=pl.DeviceIdType.MESH,
            )
            rdma_b.start()
            ca = lax.rem(d + 2 * N_DEV - 2 - s, N_DEV)
            cb = lax.rem(d + 2 + s, N_DEV)
            pa = part_a(ca)
            pb = part_b(cb)
            rdma_a.wait()
            comm_a[s + 1, :, :] = (
                comm_a[s + 1, :, :].astype(jnp.float32) + pa
            ).astype(jnp.bfloat16)
            rdma_b.wait()
            comm_b[s + 1, :, :] = (
                comm_b[s + 1, :, :].astype(jnp.float32) + pb
            ).astype(jnp.bfloat16)

        ya = comm_a[N_DEV - 1, :, :].astype(jnp.float32)
        yb = comm_b[N_DEV - 1, :, :].astype(jnp.float32)
        o_ref[:, :TILE_N] = ya * jax.nn.sigmoid(ya)
        o_ref[:, TILE_N:] = yb * jax.nn.sigmoid(yb)

    out = pl.pallas_call(
        body,
        grid=(N_STEPS,),
        in_specs=[
            pl.BlockSpec((M, K_SHARD), lambda t: (0, 0)),
            pl.BlockSpec((K_SHARD, STEP_N), lambda t: (0, t)),
        ],
        out_specs=pl.BlockSpec((M_CHUNK, STEP_N), lambda t: (0, t)),
        out_shape=jax.ShapeDtypeStruct((M_CHUNK, N_GLOBAL), jnp.float32),
        scratch_shapes=[
            pltpu.VMEM((N_DEV, M_CHUNK, TILE_N), jnp.bfloat16),
            pltpu.VMEM((N_DEV, M_CHUNK, TILE_N), jnp.bfloat16),
            pltpu.SemaphoreType.DMA((N_DEV - 1,)),
            pltpu.SemaphoreType.DMA((N_DEV - 1,)),
            pltpu.SemaphoreType.DMA((N_DEV - 1,)),
            pltpu.SemaphoreType.DMA((N_DEV - 1,)),
        ],
        compiler_params=pltpu.CompilerParams(
            collective_id=0,
            dimension_semantics=("arbitrary",),
            vmem_limit_bytes=60 * 1024 * 1024,
        ),
    )(x.astype(jnp.bfloat16), w_mat.astype(jnp.bfloat16))
    return out


# device time: 378335 ns/iter; 3.2923x vs baseline; 1.0402x over previous
import jax
import jax.numpy as jnp
from jax import lax
from jax.experimental import pallas as pl
from jax.experimental.pallas import tpu as pltpu

N_DEV = 4
M = 4096
K_SHARD = 1024
N_GLOBAL = 8192
M_CHUNK = M // N_DEV
TILE_N = 1024
STEP_N = 2 * TILE_N
N_STEPS = N_GLOBAL // STEP_N


def kernel(x, w_mat):
    def body(x_ref, w_ref, o_ref, comm_a, comm_b,
             send_a, recv_a, send_b, recv_b):
        t = pl.program_id(0)
        d = lax.axis_index("i")
        left = lax.rem(d + N_DEV - 1, N_DEV)
        right = lax.rem(d + 1, N_DEV)
        col_a = t * STEP_N
        col_b = col_a + TILE_N

        barrier = pltpu.get_barrier_semaphore()
        pl.semaphore_signal(
            barrier, inc=1, device_id=(left,),
            device_id_type=pl.DeviceIdType.MESH,
        )
        pl.semaphore_signal(
            barrier, inc=1, device_id=(right,),
            device_id_type=pl.DeviceIdType.MESH,
        )
        pl.semaphore_wait(barrier, 2)

        def part(c, col):
            return jnp.dot(
                x_ref[pl.ds(c * M_CHUNK, M_CHUNK), :],
                w_ref[:, pl.ds(col, TILE_N)],
                preferred_element_type=jnp.float32,
            )

        @pl.when(t == 0)
        def _():
            comm_a[0, :, :] = part(lax.rem(d + N_DEV - 1, N_DEV),
                                   col_a).astype(jnp.bfloat16)
            comm_b[0, :, :] = part(lax.rem(d + 1, N_DEV),
                                   col_b).astype(jnp.bfloat16)

        for s in range(N_DEV - 1):
            rdma_a = pltpu.make_async_remote_copy(
                src_ref=comm_a.at[s],
                dst_ref=comm_a.at[s + 1],
                send_sem=send_a.at[s],
                recv_sem=recv_a.at[s],
                device_id=(right,),
                device_id_type=pl.DeviceIdType.MESH,
            )
            rdma_a.start()
            rdma_b = pltpu.make_async_remote_copy(
                src_ref=comm_b.at[s],
                dst_ref=comm_b.at[s + 1],
                send_sem=send_b.at[s],
                recv_sem=recv_b.at[s],
                device_id=(left,),
                device_id_type=pl.DeviceIdType.MESH,
            )
            rdma_b.start()
            ca = lax.rem(d + 2 * N_DEV - 2 - s, N_DEV)
            cb = lax.rem(d + 2 + s, N_DEV)
            pa = part(ca, col_a).astype(jnp.bfloat16)
            pb = part(cb, col_b).astype(jnp.bfloat16)
            if s == 1:
                @pl.when(t + 1 < N_STEPS)
                def _():
                    comm_a[0, :, :] = part(
                        lax.rem(d + N_DEV - 1, N_DEV),
                        col_a + STEP_N).astype(jnp.bfloat16)
                    comm_b[0, :, :] = part(
                        lax.rem(d + 1, N_DEV),
                        col_b + STEP_N).astype(jnp.bfloat16)
            rdma_a.wait()
            comm_a[s + 1, :, :] = (
                comm_a[s + 1, :, :].astype(jnp.float32)
                + pa.astype(jnp.float32)
            ).astype(jnp.bfloat16)
            rdma_b.wait()
            comm_b[s + 1, :, :] = (
                comm_b[s + 1, :, :].astype(jnp.float32)
                + pb.astype(jnp.float32)
            ).astype(jnp.bfloat16)

        ya = comm_a[N_DEV - 1, :, :].astype(jnp.float32)
        yb = comm_b[N_DEV - 1, :, :].astype(jnp.float32)
        o_ref[:, :TILE_N] = (ya * jax.nn.sigmoid(ya)).astype(jnp.bfloat16)
        o_ref[:, TILE_N:] = (yb * jax.nn.sigmoid(yb)).astype(jnp.bfloat16)

    out = pl.pallas_call(
        body,
        grid=(N_STEPS,),
        in_specs=[
            pl.BlockSpec((M, K_SHARD), lambda t: (0, 0)),
            pl.BlockSpec((K_SHARD, N_GLOBAL), lambda t: (0, 0)),
        ],
        out_specs=pl.BlockSpec((M_CHUNK, STEP_N), lambda t: (0, t)),
        out_shape=jax.ShapeDtypeStruct((M_CHUNK, N_GLOBAL), jnp.bfloat16),
        scratch_shapes=[
            pltpu.VMEM((N_DEV, M_CHUNK, TILE_N), jnp.bfloat16),
            pltpu.VMEM((N_DEV, M_CHUNK, TILE_N), jnp.bfloat16),
            pltpu.SemaphoreType.DMA((N_DEV - 1,)),
            pltpu.SemaphoreType.DMA((N_DEV - 1,)),
            pltpu.SemaphoreType.DMA((N_DEV - 1,)),
            pltpu.SemaphoreType.DMA((N_DEV - 1,)),
        ],
        compiler_params=pltpu.CompilerParams(
            collective_id=0,
            dimension_semantics=("arbitrary",),
            vmem_limit_bytes=60 * 1024 * 1024,
        ),
    )(x.astype(jnp.bfloat16), w_mat.astype(jnp.bfloat16))
    return out.astype(jnp.float32)
